# Initial kernel scaffold; baseline (speedup 1.0000x reference)
import jax
import jax.numpy as jnp
from jax.experimental import pallas as pl


def kernel(input, target):
    raise NotImplementedError("write your pallas kernel here")


# speedup vs baseline: 2.1985x; 2.1985x over previous
import functools

import jax
import jax.numpy as jnp
from jax import lax
from jax.experimental import pallas as pl
from jax.experimental.pallas import tpu as pltpu
from jax.experimental.pallas import tpu_sc as plsc

_L = 16
_B = 4
_C = 192
_P = 224 * 224
_NTILES = 32
_TPB = _NTILES // _B
_CPT = _C // _TPB
_Q = 1792
_NCHUNK = _P // _Q
_NVEC = _Q // _L


def _body(x_hbm, t_hbm, out_hbm, t_buf, x_buf, n_acc, s_acc, sq_acc, stage):
    cid = lax.axis_index("c")
    sid = lax.axis_index("s")
    wid = cid * 16 + sid
    b = wid // _TPB
    c0 = pl.multiple_of((wid % _TPB) * _CPT, _CPT)

    zeros = jnp.zeros((_L,), jnp.float32)
    ones = jnp.ones((_L,), jnp.float32)
    n_acc[...] = zeros
    for c in range(_CPT):
        s_acc[pl.ds(c * _L, _L)] = zeros
        sq_acc[pl.ds(c * _L, _L)] = zeros

    def chunk_body(j, carry):
        p0 = pl.multiple_of(j * _Q, _Q)
        t0 = pl.multiple_of(b * _P + j * _Q, _Q)
        pltpu.sync_copy(t_hbm.at[pl.ds(t0, _Q)], t_buf)
        pltpu.sync_copy(x_hbm.at[b, pl.ds(c0, _CPT), pl.ds(p0, _Q)], x_buf)

        def vec_body(v, carry2):
            off = pl.multiple_of(v * _L, _L)
            t_vec = t_buf[pl.ds(off, _L)]
            plsc.addupdate_scatter(n_acc, [t_vec], ones)
            for c in range(_CPT):
                xv = x_buf[c, pl.ds(off, _L)]
                idx = t_vec + (c * _L)
                plsc.addupdate_scatter(s_acc, [idx], xv)
                plsc.addupdate_scatter(sq_acc, [idx], xv * xv)
            return carry2

        lax.fori_loop(0, _NVEC, vec_body, 0)
        return carry

    lax.fori_loop(0, _NCHUNK, chunk_body, 0)

    nv = n_acc[...]
    labels = lax.iota(jnp.int32, 16)
    safe_n = jnp.maximum(nv, 1.0)
    denom = jnp.maximum(nv - 1.0, 1.0)
    valid = (labels != 0) & (nv > 1.0)
    present = (labels != 0) & (nv > 0.0)
    nu = jnp.sum(present.astype(jnp.float32))
    acc = zeros
    for c in range(_CPT):
        s = s_acc[pl.ds(c * _L, _L)]
        sq = sq_acc[pl.ds(c * _L, _L)]
        mean = s / safe_n
        acc = acc + (sq - nv * mean * mean) / denom
    acc = jnp.where(valid, acc, 0.0)
    nu_vec = jnp.full((_L,), nu, jnp.float32) + 1e-8
    acc = acc / nu_vec
    q = jnp.sum(acc)
    stage[...] = jnp.where(labels == 0, q, 0.0)
    pltpu.sync_copy(stage, out_hbm.at[wid])


def kernel(input, target):
    x = input.reshape(_B, _C, _P)
    t = target.reshape(_B * _P)
    mesh = plsc.VectorSubcoreMesh(core_axis_name="c", subcore_axis_name="s")
    run = pl.kernel(
        _body,
        out_type=jax.ShapeDtypeStruct((_NTILES, _L), jnp.float32),
        mesh=mesh,
        compiler_params=pltpu.CompilerParams(needs_layout_passes=False),
        scratch_types=[
            pltpu.VMEM((_Q,), jnp.int32),
            pltpu.VMEM((_CPT, _Q), jnp.float32),
            pltpu.VMEM((_L,), jnp.float32),
            pltpu.VMEM((_CPT * _L,), jnp.float32),
            pltpu.VMEM((_CPT * _L,), jnp.float32),
            pltpu.VMEM((_L,), jnp.float32),
        ],
    )
    out = run(x, t)
    return jnp.sum(out) / _B


# speedup vs baseline: 2.4500x; 1.1144x over previous
import functools

import jax
import jax.numpy as jnp
from jax import lax
from jax.experimental import pallas as pl
from jax.experimental.pallas import tpu as pltpu
from jax.experimental.pallas import tpu_sc as plsc

_L = 16
_B = 4
_C = 192
_P = 224 * 224
_NTILES = 32
_TPB = _NTILES // _B
_CPT = _C // _TPB
_Q = 1792
_NCHUNK = _P // _Q
_NVEC = _Q // _L


def _body(x_hbm, t_hbm, out_hbm, t_buf, x_buf, n_acc, s_acc, sq_acc, stage):
    cid = lax.axis_index("c")
    sid = lax.axis_index("s")
    wid = cid * 16 + sid
    b = wid // _TPB
    c0 = pl.multiple_of((wid % _TPB) * _CPT, _CPT)

    zeros = jnp.zeros((_L,), jnp.float32)
    ones = jnp.ones((_L,), jnp.float32)
    bank_n = lax.iota(jnp.int32, _L) * _L
    bank_s = lax.iota(jnp.int32, _L) * (_CPT * _L)

    def init_body(k, carry):
        o = pl.multiple_of(k * _L, _L)
        n_acc[pl.ds(o, _L)] = zeros
        return carry

    lax.fori_loop(0, _L, init_body, 0)

    def init_body2(k, carry):
        o = pl.multiple_of(k * _L, _L)
        s_acc[pl.ds(o, _L)] = zeros
        sq_acc[pl.ds(o, _L)] = zeros
        return carry

    lax.fori_loop(0, _L * _CPT, init_body2, 0)

    def chunk_body(j, carry):
        p0 = pl.multiple_of(j * _Q, _Q)
        t0 = pl.multiple_of(b * _P + j * _Q, _Q)
        pltpu.sync_copy(t_hbm.at[pl.ds(t0, _Q)], t_buf)
        pltpu.sync_copy(x_hbm.at[b, pl.ds(c0, _CPT), pl.ds(p0, _Q)], x_buf)

        def vec_body(v, carry2):
            off = pl.multiple_of(v * _L, _L)
            t_vec = t_buf[pl.ds(off, _L)]
            plsc.addupdate_scatter(n_acc, [t_vec + bank_n], ones)
            tb = t_vec + bank_s
            for c in range(_CPT):
                xv = x_buf[c, pl.ds(off, _L)]
                idx = tb + (c * _L)
                plsc.addupdate_scatter(s_acc, [idx], xv)
                plsc.addupdate_scatter(sq_acc, [idx], xv * xv)
            return carry2

        lax.fori_loop(0, _NVEC, vec_body, 0)
        return carry

    lax.fori_loop(0, _NCHUNK, chunk_body, 0)

    nv = zeros
    for k in range(_L):
        nv = nv + n_acc[pl.ds(k * _L, _L)]
    labels = lax.iota(jnp.int32, 16)
    safe_n = jnp.maximum(nv, 1.0)
    denom = jnp.maximum(nv - 1.0, 1.0)
    valid = (labels != 0) & (nv > 1.0)
    present = (labels != 0) & (nv > 0.0)
    nu = jnp.sum(present.astype(jnp.float32))
    acc = zeros
    for c in range(_CPT):
        s = zeros
        sq = zeros
        for k in range(_L):
            o = k * _CPT * _L + c * _L
            s = s + s_acc[pl.ds(o, _L)]
            sq = sq + sq_acc[pl.ds(o, _L)]
        mean = s / safe_n
        acc = acc + (sq - nv * mean * mean) / denom
    acc = jnp.where(valid, acc, 0.0)
    nu_vec = jnp.full((_L,), nu, jnp.float32) + 1e-8
    acc = acc / nu_vec
    q = jnp.sum(acc)
    stage[...] = jnp.where(labels == 0, q, 0.0)
    pltpu.sync_copy(stage, out_hbm.at[wid])


def kernel(input, target):
    x = input.reshape(_B, _C, _P)
    t = target.reshape(_B * _P)
    mesh = plsc.VectorSubcoreMesh(core_axis_name="c", subcore_axis_name="s")
    run = pl.kernel(
        _body,
        out_type=jax.ShapeDtypeStruct((_NTILES, _L), jnp.float32),
        mesh=mesh,
        compiler_params=pltpu.CompilerParams(needs_layout_passes=False),
        scratch_types=[
            pltpu.VMEM((_Q,), jnp.int32),
            pltpu.VMEM((_CPT, _Q), jnp.float32),
            pltpu.VMEM((_L * _L,), jnp.float32),
            pltpu.VMEM((_L * _CPT * _L,), jnp.float32),
            pltpu.VMEM((_L * _CPT * _L,), jnp.float32),
            pltpu.VMEM((_L,), jnp.float32),
        ],
    )
    out = run(x, t)
    return jnp.sum(out) / _B


# speedup vs baseline: 2.7114x; 1.1067x over previous
import functools

import jax
import jax.numpy as jnp
from jax import lax
from jax.experimental import pallas as pl
from jax.experimental.pallas import tpu as pltpu
from jax.experimental.pallas import tpu_sc as plsc

_L = 16
_B = 4
_C = 192
_P = 224 * 224
_NTILES = 32
_TPB = _NTILES // _B
_CPT = _C // _TPB
_Q = 1792
_NCHUNK = _P // _Q
_NVEC = _Q // _L


def _body(x_hbm, t_hbm, out_hbm, t_buf, x_buf, n_acc, s_acc, sq_acc, stage,
          sem_t0, sem_x0, sem_t1, sem_x1):
    cid = lax.axis_index("c")
    sid = lax.axis_index("s")
    wid = cid * 16 + sid
    b = wid // _TPB
    c0 = pl.multiple_of((wid % _TPB) * _CPT, _CPT)

    zeros = jnp.zeros((_L,), jnp.float32)
    ones = jnp.ones((_L,), jnp.float32)
    bank_n = lax.iota(jnp.int32, _L) * _L
    bank_s = lax.iota(jnp.int32, _L) * (_CPT * _L)

    def init_body(k, carry):
        o = pl.multiple_of(k * _L, _L)
        n_acc[pl.ds(o, _L)] = zeros
        return carry

    lax.fori_loop(0, _L, init_body, 0)

    def init_body2(k, carry):
        o = pl.multiple_of(k * _L, _L)
        s_acc[pl.ds(o, _L)] = zeros
        sq_acc[pl.ds(o, _L)] = zeros
        return carry

    lax.fori_loop(0, _L * _CPT, init_body2, 0)

    sem_t = (sem_t0, sem_t1)
    sem_x = (sem_x0, sem_x1)

    def start_chunk(j, slot):
        p0 = pl.multiple_of(j * _Q, _Q)
        t0 = pl.multiple_of(b * _P + j * _Q, _Q)
        pltpu.async_copy(t_hbm.at[pl.ds(t0, _Q)], t_buf.at[slot], sem_t[slot])
        pltpu.async_copy(
            x_hbm.at[b, pl.ds(c0, _CPT), pl.ds(p0, _Q)], x_buf.at[slot],
            sem_x[slot])

    def wait_chunk(slot):
        pltpu.make_async_copy(
            t_hbm.at[pl.ds(0, _Q)], t_buf.at[slot], sem_t[slot]).wait()
        pltpu.make_async_copy(
            x_hbm.at[b, pl.ds(0, _CPT), pl.ds(0, _Q)], x_buf.at[slot],
            sem_x[slot]).wait()

    def compute_chunk(slot):
        def vec_body(v, carry2):
            off = pl.multiple_of(v * _L, _L)
            t_vec = t_buf[slot, pl.ds(off, _L)]
            plsc.addupdate_scatter(n_acc, [t_vec + bank_n], ones)
            tb = t_vec + bank_s
            for c in range(_CPT):
                xv = x_buf[slot, c, pl.ds(off, _L)]
                idx = tb + (c * _L)
                plsc.addupdate_scatter(s_acc, [idx], xv)
                plsc.addupdate_scatter(sq_acc, [idx], xv * xv)
            return carry2

        lax.fori_loop(0, _NVEC, vec_body, 0)

    start_chunk(0, 0)

    def pair_body(g, carry):
        base = g * 2
        start_chunk(base + 1, 1)
        wait_chunk(0)
        compute_chunk(0)

        @pl.when(base + 2 < _NCHUNK)
        def _():
            start_chunk(base + 2, 0)

        wait_chunk(1)
        compute_chunk(1)
        return carry

    lax.fori_loop(0, _NCHUNK // 2, pair_body, 0)

    nv = zeros
    for k in range(_L):
        nv = nv + n_acc[pl.ds(k * _L, _L)]
    labels = lax.iota(jnp.int32, 16)
    safe_n = jnp.maximum(nv, 1.0)
    denom = jnp.maximum(nv - 1.0, 1.0)
    valid = (labels != 0) & (nv > 1.0)
    present = (labels != 0) & (nv > 0.0)
    nu = jnp.sum(present.astype(jnp.float32))
    acc = zeros
    for c in range(_CPT):
        s = zeros
        sq = zeros
        for k in range(_L):
            o = k * _CPT * _L + c * _L
            s = s + s_acc[pl.ds(o, _L)]
            sq = sq + sq_acc[pl.ds(o, _L)]
        mean = s / safe_n
        acc = acc + (sq - nv * mean * mean) / denom
    acc = jnp.where(valid, acc, 0.0)
    nu_vec = jnp.full((_L,), nu, jnp.float32) + 1e-8
    acc = acc / nu_vec
    q = jnp.sum(acc)
    stage[...] = jnp.where(labels == 0, q, 0.0)
    pltpu.sync_copy(stage, out_hbm.at[wid])


def kernel(input, target):
    x = input.reshape(_B, _C, _P)
    t = target.reshape(_B * _P)
    mesh = plsc.VectorSubcoreMesh(core_axis_name="c", subcore_axis_name="s")
    run = pl.kernel(
        _body,
        out_type=jax.ShapeDtypeStruct((_NTILES, _L), jnp.float32),
        mesh=mesh,
        compiler_params=pltpu.CompilerParams(needs_layout_passes=False),
        scratch_types=[
            pltpu.VMEM((2, _Q), jnp.int32),
            pltpu.VMEM((2, _CPT, _Q), jnp.float32),
            pltpu.VMEM((_L * _L,), jnp.float32),
            pltpu.VMEM((_L * _CPT * _L,), jnp.float32),
            pltpu.VMEM((_L * _CPT * _L,), jnp.float32),
            pltpu.VMEM((_L,), jnp.float32),
            pltpu.SemaphoreType.DMA,
            pltpu.SemaphoreType.DMA,
            pltpu.SemaphoreType.DMA,
            pltpu.SemaphoreType.DMA,
        ],
    )
    out = run(x, t)
    return jnp.sum(out) / _B


# speedup vs baseline: 4.0794x; 1.5046x over previous
import functools

import jax
import jax.numpy as jnp
from jax import lax
from jax.experimental import pallas as pl
from jax.experimental.pallas import tpu as pltpu
from jax.experimental.pallas import tpu_sc as plsc

_L = 16
_B = 4
_C = 192
_P = 224 * 224
_NTILES = 32
_TPB = _NTILES // _B
_CPT = _C // _TPB
_Q = 1792
_NCHUNK = _P // _Q
_NVEC = _Q // _L


def _body(x_hbm, t_hbm, out_hbm, t_buf, x_buf, n_acc, s_acc, sq_acc, stage,
          sem_t0, sem_x0, sem_t1, sem_x1):
    cid = lax.axis_index("c")
    sid = lax.axis_index("s")
    wid = cid * 16 + sid
    b = wid // _TPB
    c0 = pl.multiple_of((wid % _TPB) * _CPT, _CPT)

    zeros = jnp.zeros((_L,), jnp.float32)
    ones = jnp.ones((_L,), jnp.float32)
    bank_n = lax.iota(jnp.int32, _L) * _L

    def init_body(k, carry):
        o = pl.multiple_of(k * _L, _L)
        n_acc[pl.ds(o, _L)] = zeros
        return carry

    lax.fori_loop(0, _L, init_body, 0)

    def init_body2(k, carry):
        o = pl.multiple_of(k * _L, _L)
        s_acc[pl.ds(o, _L)] = zeros
        sq_acc[pl.ds(o, _L)] = zeros
        return carry

    lax.fori_loop(0, _L * _CPT, init_body2, 0)

    sem_t = (sem_t0, sem_t1)
    sem_x = (sem_x0, sem_x1)

    def start_chunk(j, slot):
        p0 = pl.multiple_of(j * _Q, _Q)
        t0 = pl.multiple_of(b * _P + j * _Q, _Q)
        pltpu.async_copy(t_hbm.at[pl.ds(t0, _Q)], t_buf.at[slot], sem_t[slot])
        pltpu.async_copy(
            x_hbm.at[b, pl.ds(c0, _CPT), pl.ds(p0, _Q)], x_buf.at[slot],
            sem_x[slot])

    def wait_chunk(slot):
        pltpu.make_async_copy(
            t_hbm.at[pl.ds(0, _Q)], t_buf.at[slot], sem_t[slot]).wait()
        pltpu.make_async_copy(
            x_hbm.at[b, pl.ds(0, _CPT), pl.ds(0, _Q)], x_buf.at[slot],
            sem_x[slot]).wait()

    def compute_chunk(slot):
        def vec_body(v, carry2):
            off = pl.multiple_of(v * _L, _L)
            t_vec = t_buf[slot, pl.ds(off, _L)]
            tb = t_vec + bank_n
            plsc.addupdate_scatter(n_acc, [tb], ones)
            for g in range(_CPT // 8):
                xs = [x_buf[slot, g * 8 + u, pl.ds(off, _L)]
                      for u in range(8)]
                for u in range(8):
                    c = g * 8 + u
                    blk_s = s_acc.at[pl.ds(c * _L * _L, _L * _L)]
                    blk_q = sq_acc.at[pl.ds(c * _L * _L, _L * _L)]
                    plsc.addupdate_scatter(blk_s, [tb], xs[u])
                    plsc.addupdate_scatter(blk_q, [tb], xs[u] * xs[u])
            return carry2

        lax.fori_loop(0, _NVEC, vec_body, 0)

    start_chunk(0, 0)

    def pair_body(g, carry):
        base = g * 2
        start_chunk(base + 1, 1)
        wait_chunk(0)
        compute_chunk(0)

        @pl.when(base + 2 < _NCHUNK)
        def _():
            start_chunk(base + 2, 0)

        wait_chunk(1)
        compute_chunk(1)
        return carry

    lax.fori_loop(0, _NCHUNK // 2, pair_body, 0)

    nv = zeros
    for k in range(_L):
        nv = nv + n_acc[pl.ds(k * _L, _L)]
    labels = lax.iota(jnp.int32, 16)
    safe_n = jnp.maximum(nv, 1.0)
    denom = jnp.maximum(nv - 1.0, 1.0)
    valid = (labels != 0) & (nv > 1.0)
    present = (labels != 0) & (nv > 0.0)
    nu = jnp.sum(present.astype(jnp.float32))
    acc = zeros
    for c in range(_CPT):
        s = zeros
        sq = zeros
        for k in range(_L):
            o = c * _L * _L + k * _L
            s = s + s_acc[pl.ds(o, _L)]
            sq = sq + sq_acc[pl.ds(o, _L)]
        mean = s / safe_n
        acc = acc + (sq - nv * mean * mean) / denom
    acc = jnp.where(valid, acc, 0.0)
    nu_vec = jnp.full((_L,), nu, jnp.float32) + 1e-8
    acc = acc / nu_vec
    q = jnp.sum(acc)
    stage[...] = jnp.where(labels == 0, q, 0.0)
    pltpu.sync_copy(stage, out_hbm.at[wid])


def kernel(input, target):
    x = input.reshape(_B, _C, _P)
    t = target.reshape(_B * _P)
    mesh = plsc.VectorSubcoreMesh(core_axis_name="c", subcore_axis_name="s")
    run = pl.kernel(
        _body,
        out_type=jax.ShapeDtypeStruct((_NTILES, _L), jnp.float32),
        mesh=mesh,
        compiler_params=pltpu.CompilerParams(needs_layout_passes=False),
        scratch_types=[
            pltpu.VMEM((2, _Q), jnp.int32),
            pltpu.VMEM((2, _CPT, _Q), jnp.float32),
            pltpu.VMEM((_L * _L,), jnp.float32),
            pltpu.VMEM((_L * _CPT * _L,), jnp.float32),
            pltpu.VMEM((_L * _CPT * _L,), jnp.float32),
            pltpu.VMEM((_L,), jnp.float32),
            pltpu.SemaphoreType.DMA,
            pltpu.SemaphoreType.DMA,
            pltpu.SemaphoreType.DMA,
            pltpu.SemaphoreType.DMA,
        ],
    )
    out = run(x, t)
    return jnp.sum(out) / _B


# speedup vs baseline: 5.4766x; 1.3425x over previous
import functools

import jax
import jax.numpy as jnp
from jax import lax
from jax.experimental import pallas as pl
from jax.experimental.pallas import tpu as pltpu
from jax.experimental.pallas import tpu_sc as plsc

_L = 16
_B = 4
_C = 192
_P = 224 * 224
_NTILES = 32
_TPB = _NTILES // _B
_CPT = _C // _TPB
_Q = 1792
_NCHUNK = _P // _Q
_NVEC = _Q // _L


def _body(x_hbm, t_hbm, out_hbm, t_buf, x_buf, n_acc, s_acc, sq_acc, stage,
          sem_t0, sem_x0, sem_t1, sem_x1):
    cid = lax.axis_index("c")
    sid = lax.axis_index("s")
    wid = cid * 16 + sid
    b = wid // _TPB
    c0 = pl.multiple_of((wid % _TPB) * _CPT, _CPT)

    zeros = jnp.zeros((_L,), jnp.float32)
    ones = jnp.ones((_L,), jnp.float32)
    lane = lax.iota(jnp.int32, _L)

    def init_body(k, carry):
        o = pl.multiple_of(k * _L, _L)
        n_acc[pl.ds(o, _L)] = zeros
        return carry

    lax.fori_loop(0, _L, init_body, 0)

    def init_body2(k, carry):
        o = pl.multiple_of(k * _L, _L)
        s_acc[pl.ds(o, _L)] = zeros
        sq_acc[pl.ds(o, _L)] = zeros
        return carry

    lax.fori_loop(0, _L * _CPT, init_body2, 0)

    sem_t = (sem_t0, sem_t1)
    sem_x = (sem_x0, sem_x1)

    def start_chunk(j, slot):
        p0 = pl.multiple_of(j * _Q, _Q)
        t0 = pl.multiple_of(b * _P + j * _Q, _Q)
        pltpu.async_copy(t_hbm.at[pl.ds(t0, _Q)], t_buf.at[slot], sem_t[slot])
        pltpu.async_copy(
            x_hbm.at[b, pl.ds(c0, _CPT), pl.ds(p0, _Q)], x_buf.at[slot],
            sem_x[slot])

    def wait_chunk(slot):
        pltpu.make_async_copy(
            t_hbm.at[pl.ds(0, _Q)], t_buf.at[slot], sem_t[slot]).wait()
        pltpu.make_async_copy(
            x_hbm.at[b, pl.ds(0, _CPT), pl.ds(0, _Q)], x_buf.at[slot],
            sem_x[slot]).wait()

    def compute_chunk(slot):
        def vec_body(v, carry2):
            off = pl.multiple_of(v * _L, _L)
            t_vec = t_buf[slot, pl.ds(off, _L)]
            tb = t_vec * _L + lane
            plsc.addupdate_scatter(n_acc, [tb], ones)
            for g in range(_CPT // 8):
                xs = [x_buf[slot, g * 8 + u, pl.ds(off, _L)]
                      for u in range(8)]
                for u in range(8):
                    c = g * 8 + u
                    blk_s = s_acc.at[pl.ds(c * _L * _L, _L * _L)]
                    blk_q = sq_acc.at[pl.ds(c * _L * _L, _L * _L)]
                    plsc.addupdate_scatter(blk_s, [tb], xs[u])
                    plsc.addupdate_scatter(blk_q, [tb], xs[u] * xs[u])
            return carry2

        lax.fori_loop(0, _NVEC, vec_body, 0)

    start_chunk(0, 0)

    def pair_body(g, carry):
        base = g * 2
        start_chunk(base + 1, 1)
        wait_chunk(0)
        compute_chunk(0)

        @pl.when(base + 2 < _NCHUNK)
        def _():
            start_chunk(base + 2, 0)

        wait_chunk(1)
        compute_chunk(1)
        return carry

    lax.fori_loop(0, _NCHUNK // 2, pair_body, 0)

    def bank_sum(ref, base):
        col = base + lane * _L
        tot = zeros
        for k in range(_L):
            tot = tot + plsc.load_gather(ref, [col + k])
        return tot

    nv = bank_sum(n_acc, 0)
    labels = lax.iota(jnp.int32, 16)
    safe_n = jnp.maximum(nv, 1.0)
    denom = jnp.maximum(nv - 1.0, 1.0)
    valid = (labels != 0) & (nv > 1.0)
    present = (labels != 0) & (nv > 0.0)
    nu = jnp.sum(present.astype(jnp.float32))
    acc = zeros
    for c in range(_CPT):
        s = bank_sum(s_acc, c * _L * _L)
        sq = bank_sum(sq_acc, c * _L * _L)
        mean = s / safe_n
        acc = acc + (sq - nv * mean * mean) / denom
    acc = jnp.where(valid, acc, 0.0)
    nu_vec = jnp.full((_L,), nu, jnp.float32) + 1e-8
    acc = acc / nu_vec
    q = jnp.sum(acc)
    stage[...] = jnp.where(labels == 0, q, 0.0)
    pltpu.sync_copy(stage, out_hbm.at[wid])


def kernel(input, target):
    x = input.reshape(_B, _C, _P)
    t = target.reshape(_B * _P)
    mesh = plsc.VectorSubcoreMesh(core_axis_name="c", subcore_axis_name="s")
    run = pl.kernel(
        _body,
        out_type=jax.ShapeDtypeStruct((_NTILES, _L), jnp.float32),
        mesh=mesh,
        compiler_params=pltpu.CompilerParams(needs_layout_passes=False),
        scratch_types=[
            pltpu.VMEM((2, _Q), jnp.int32),
            pltpu.VMEM((2, _CPT, _Q), jnp.float32),
            pltpu.VMEM((_L * _L,), jnp.float32),
            pltpu.VMEM((_L * _CPT * _L,), jnp.float32),
            pltpu.VMEM((_L * _CPT * _L,), jnp.float32),
            pltpu.VMEM((_L,), jnp.float32),
            pltpu.SemaphoreType.DMA,
            pltpu.SemaphoreType.DMA,
            pltpu.SemaphoreType.DMA,
            pltpu.SemaphoreType.DMA,
        ],
    )
    out = run(x, t)
    return jnp.sum(out) / _B


# speedup vs baseline: 5.5651x; 1.0162x over previous
import functools

import jax
import jax.numpy as jnp
from jax import lax
from jax.experimental import pallas as pl
from jax.experimental.pallas import tpu as pltpu
from jax.experimental.pallas import tpu_sc as plsc

_L = 16
_B = 4
_C = 192
_P = 224 * 224
_NTILES = 32
_TPB = _NTILES // _B
_CPT = _C // _TPB
_Q = 1792
_NCHUNK = _P // _Q
_NVEC = _Q // _L


def _body(x_hbm, t_hbm, out_hbm, t_buf, x_buf, n_acc, s_acc, sq_acc, stage,
          sem_t0, sem_x0, sem_t1, sem_x1):
    cid = lax.axis_index("c")
    sid = lax.axis_index("s")
    wid = cid * 16 + sid
    b = wid // _TPB
    c0 = pl.multiple_of((wid % _TPB) * _CPT, _CPT)

    zeros = jnp.zeros((_L,), jnp.float32)
    ones = jnp.ones((_L,), jnp.float32)
    lane = lax.iota(jnp.int32, _L)

    def init_body(k, carry):
        o = pl.multiple_of(k * _L, _L)
        n_acc[pl.ds(o, _L)] = zeros
        return carry

    lax.fori_loop(0, _L, init_body, 0)

    def init_body2(k, carry):
        o = pl.multiple_of(k * _L, _L)
        s_acc[pl.ds(o, _L)] = zeros
        sq_acc[pl.ds(o, _L)] = zeros
        return carry

    lax.fori_loop(0, _L * _CPT, init_body2, 0)

    sem_t = (sem_t0, sem_t1)
    sem_x = (sem_x0, sem_x1)

    def start_chunk(j, slot):
        p0 = pl.multiple_of(j * _Q, _Q)
        t0 = pl.multiple_of(b * _P + j * _Q, _Q)
        pltpu.async_copy(t_hbm.at[pl.ds(t0, _Q)], t_buf.at[slot], sem_t[slot])
        pltpu.async_copy(
            x_hbm.at[b, pl.ds(c0, _CPT), pl.ds(p0, _Q)], x_buf.at[slot],
            sem_x[slot])

    def wait_chunk(slot):
        pltpu.make_async_copy(
            t_hbm.at[pl.ds(0, _Q)], t_buf.at[slot], sem_t[slot]).wait()
        pltpu.make_async_copy(
            x_hbm.at[b, pl.ds(0, _CPT), pl.ds(0, _Q)], x_buf.at[slot],
            sem_x[slot]).wait()

    def compute_chunk(slot):
        def vec_body(v, carry2):
            off = pl.multiple_of(v * (2 * _L), 2 * _L)
            tbs = []
            for h in range(2):
                t_vec = t_buf[slot, pl.ds(off + h * _L, _L)]
                tbs.append(t_vec * _L + lane)
            for h in range(2):
                plsc.addupdate_scatter(n_acc, [tbs[h]], ones)
            for h in range(2):
                tb = tbs[h]
                for g in range(_CPT // 8):
                    xs = [x_buf[slot, g * 8 + u, pl.ds(off + h * _L, _L)]
                          for u in range(8)]
                    for u in range(8):
                        c = g * 8 + u
                        blk_s = s_acc.at[pl.ds(c * _L * _L, _L * _L)]
                        blk_q = sq_acc.at[pl.ds(c * _L * _L, _L * _L)]
                        plsc.addupdate_scatter(blk_s, [tb], xs[u])
                        plsc.addupdate_scatter(blk_q, [tb], xs[u] * xs[u])
            return carry2

        lax.fori_loop(0, _NVEC // 2, vec_body, 0)

    start_chunk(0, 0)

    def pair_body(g, carry):
        base = g * 2
        start_chunk(base + 1, 1)
        wait_chunk(0)
        compute_chunk(0)

        @pl.when(base + 2 < _NCHUNK)
        def _():
            start_chunk(base + 2, 0)

        wait_chunk(1)
        compute_chunk(1)
        return carry

    lax.fori_loop(0, _NCHUNK // 2, pair_body, 0)

    def bank_sum(ref, base):
        col = base + lane * _L
        tot = zeros
        for k in range(_L):
            tot = tot + plsc.load_gather(ref, [col + k])
        return tot

    nv = bank_sum(n_acc, 0)
    labels = lax.iota(jnp.int32, 16)
    safe_n = jnp.maximum(nv, 1.0)
    denom = jnp.maximum(nv - 1.0, 1.0)
    valid = (labels != 0) & (nv > 1.0)
    present = (labels != 0) & (nv > 0.0)
    nu = jnp.sum(present.astype(jnp.float32))
    acc = zeros
    for c in range(_CPT):
        s = bank_sum(s_acc, c * _L * _L)
        sq = bank_sum(sq_acc, c * _L * _L)
        mean = s / safe_n
        acc = acc + (sq - nv * mean * mean) / denom
    acc = jnp.where(valid, acc, 0.0)
    nu_vec = jnp.full((_L,), nu, jnp.float32) + 1e-8
    acc = acc / nu_vec
    q = jnp.sum(acc)
    stage[...] = jnp.where(labels == 0, q, 0.0)
    pltpu.sync_copy(stage, out_hbm.at[wid])


def kernel(input, target):
    x = input.reshape(_B, _C, _P)
    t = target.reshape(_B * _P)
    mesh = plsc.VectorSubcoreMesh(core_axis_name="c", subcore_axis_name="s")
    run = pl.kernel(
        _body,
        out_type=jax.ShapeDtypeStruct((_NTILES, _L), jnp.float32),
        mesh=mesh,
        compiler_params=pltpu.CompilerParams(needs_layout_passes=False),
        scratch_types=[
            pltpu.VMEM((2, _Q), jnp.int32),
            pltpu.VMEM((2, _CPT, _Q), jnp.float32),
            pltpu.VMEM((_L * _L,), jnp.float32),
            pltpu.VMEM((_L * _CPT * _L,), jnp.float32),
            pltpu.VMEM((_L * _CPT * _L,), jnp.float32),
            pltpu.VMEM((_L,), jnp.float32),
            pltpu.SemaphoreType.DMA,
            pltpu.SemaphoreType.DMA,
            pltpu.SemaphoreType.DMA,
            pltpu.SemaphoreType.DMA,
        ],
    )
    out = run(x, t)
    return jnp.sum(out) / _B
